# baseline (speedup 1.0000x reference)
import numpy as np
import jax
import jax.numpy as jnp
from jax.experimental import pallas as pl

_ANCHORS = np.array([1.3221, 1.73145, 3.19275, 4.00944, 5.05587, 8.09892,
                     9.47112, 4.84053, 11.2364, 10.0071], dtype=np.float32)
_A = 5
_CLS = 20
_G = 19
_N = _A * _G * _G
_NP = 2048
_B = 8
_CHUNK = 1024
_NCHUNK = _NP // _CHUNK
_CONF_TH = 0.005
_NMS_TH = 0.45
_K = np.float32(_NMS_TH / (1.0 + _NMS_TH))


def _make_consts():
    i = np.arange(_NP)
    a = np.minimum(i // (_G * _G), _A - 1)
    p = i % (_G * _G)
    gx = (p % _G).astype(np.float32)
    gy = (p // _G).astype(np.float32)
    aw = _ANCHORS[2 * a]
    ah = _ANCHORS[2 * a + 1]
    return np.stack([gx, gy, aw, ah]).reshape(4, 1, _NP).astype(np.float32)


_CONSTS = _make_consts()


def _bitonic_stage(lane, k, j, key, idx, payloads):
    upper = (lane & j) == 0
    asc = (lane & k) == 0

    def partner(x):
        return jnp.where(upper, jnp.roll(x, -j, axis=1), jnp.roll(x, j, axis=1))

    pkey = partner(key)
    pidx = partner(idx)
    self_prec = (key > pkey) | ((key == pkey) & (idx < pidx))
    take = jnp.logical_xor(jnp.logical_xor(self_prec, upper),
                           jnp.logical_not(asc))
    new_key = jnp.where(take, pkey, key)
    new_idx = jnp.where(take, pidx, idx)
    new_pl = [jnp.where(take, partner(x), x) for x in payloads]
    return new_key, new_idx, new_pl


def _bitonic_sort(lane, key, idx, payloads):
    n = key.shape[1]
    k = 2
    while k <= n:
        j = k // 2
        while j >= 1:
            key, idx, payloads = _bitonic_stage(lane, k, j, key, idx, payloads)
            j //= 2
        k *= 2
    return key, idx, payloads


def _nms_body(o_ref, c_ref, cx_ref, cy_ref, w_ref, h_ref, p_ref):
    gx = c_ref[0]
    gy = c_ref[1]
    aw = c_ref[2]
    ah = c_ref[3]

    cx = (jax.nn.sigmoid(o_ref[0]) + gx) / np.float32(_G)
    cy = (jax.nn.sigmoid(o_ref[1]) + gy) / np.float32(_G)
    wv = jnp.exp(o_ref[2]) * aw / np.float32(_G)
    hv = jnp.exp(o_ref[3]) * ah / np.float32(_G)
    conf = jax.nn.sigmoid(o_ref[4])

    maxl = o_ref[5]
    for c in range(6, 5 + _CLS):
        maxl = jnp.maximum(maxl, o_ref[c])
    ssum = jnp.exp(o_ref[5] - maxl)
    for c in range(6, 5 + _CLS):
        ssum = ssum + jnp.exp(o_ref[c] - maxl)
    clsc = 1.0 / ssum

    lane = jax.lax.broadcasted_iota(jnp.int32, (_B, _NP), 1)
    valid = lane < _N
    wconf0 = jnp.where((conf > _CONF_TH) & valid, conf, -1.0)
    neg = np.float32(-3.4e38)

    skey, sidx, (scx, scy, swv, shv) = _bitonic_sort(
        lane, wconf0, lane, [cx, cy, wv, hv])

    sx1 = scx - swv / 2.0
    sx2 = scx + swv / 2.0
    sy1 = scy - shv / 2.0
    sy2 = scy + shv / 2.0
    ssa = (sx2 - sx1) * (sy2 - sy1) * _K

    kept_s = jnp.zeros((_B, _NP), jnp.float32)
    wc = skey
    lane_c = jax.lax.broadcasted_iota(jnp.int32, (_B, _CHUNK), 1)

    for c in range(_NCHUNK):
        base = c * _CHUNK
        S = _NP - base
        ax1 = sx1[:, base:]
        ax2 = sx2[:, base:]
        ay1 = sy1[:, base:]
        ay2 = sy2[:, base:]
        asa = ssa[:, base:]
        pcx = scx[:, base:base + _CHUNK]
        pcy = scy[:, base:base + _CHUNK]
        pw = swv[:, base:base + _CHUNK]
        ph = shv[:, base:base + _CHUNK]

        def step(wcs, kfs):
            ck = wcs[:, :_CHUNK]
            alive = ck > 0.0
            li = jnp.min(jnp.where(alive, lane_c, np.int32(2 ** 30)),
                         axis=1, keepdims=True)
            leader = (lane_c == li) & alive

            def ext(v):
                return jnp.max(jnp.where(leader, v, neg), axis=1, keepdims=True)

            lcx = ext(pcx)
            lcy = ext(pcy)
            lw = ext(pw)
            lh = ext(ph)
            lx1 = lcx - lw / 2.0
            lx2 = lcx + lw / 2.0
            ly1 = lcy - lh / 2.0
            ly2 = lcy + lh / 2.0
            lsa = (lx2 - lx1) * (ly2 - ly1) * _K
            iw = jnp.maximum(jnp.minimum(ax2, lx2) - jnp.maximum(ax1, lx1), 0.0)
            ih = jnp.maximum(jnp.minimum(ay2, ly2) - jnp.maximum(ay1, ly1), 0.0)
            keepalive = iw * ih <= asa + lsa
            new_wcs = jnp.where(keepalive, wcs, -1.0)
            new_ck = jnp.where(leader, -1.0, new_wcs[:, :_CHUNK])
            new_kf = jnp.maximum(kfs[:, :_CHUNK], leader.astype(jnp.float32))
            if S > _CHUNK:
                new_wcs = jnp.concatenate([new_ck, new_wcs[:, _CHUNK:]], axis=1)
                new_kfs = jnp.concatenate([new_kf, kfs[:, _CHUNK:]], axis=1)
            else:
                new_wcs = new_ck
                new_kfs = new_kf
            return new_wcs, new_kfs

        def ccond(st):
            wcs, _ = st
            return jnp.max(wcs[:, :_CHUNK]) > 0.0

        def cbody(st):
            wcs, kfs = st
            for _ in range(8):
                wcs, kfs = step(wcs, kfs)
            return wcs, kfs

        wc_suf, kept_suf = jax.lax.while_loop(
            ccond, cbody, (wc[:, base:], kept_s[:, base:]))
        if base > 0:
            wc = jnp.concatenate([wc[:, :base], wc_suf], axis=1)
            kept_s = jnp.concatenate([kept_s[:, :base], kept_suf], axis=1)
        else:
            wc = wc_suf
            kept_s = kept_suf

    ukey = (-sidx).astype(jnp.float32)
    _, _, (kept_orig,) = _bitonic_sort(lane, ukey, lane, [kept_s])
    prob = conf * clsc * kept_orig
    cx_ref[:, :] = cx
    cy_ref[:, :] = cy
    w_ref[:, :] = wv
    h_ref[:, :] = hv
    p_ref[:, :] = prob


def kernel(output):
    o = jnp.transpose(output.reshape(_B, _A, 5 + _CLS, _G * _G),
                      (2, 0, 1, 3)).reshape(5 + _CLS, _B, _N)
    o = jnp.pad(o, ((0, 0), (0, 0), (0, _NP - _N)))
    outs = pl.pallas_call(
        _nms_body,
        out_shape=[jax.ShapeDtypeStruct((_B, _NP), jnp.float32)] * 5,
    )(o, jnp.asarray(_CONSTS))
    return jnp.stack(outs, axis=-1)[:, :_N, :]


# speedup vs baseline: 1.0058x; 1.0058x over previous
import numpy as np
import jax
import jax.numpy as jnp
from jax.experimental import pallas as pl

_ANCHORS = np.array([1.3221, 1.73145, 3.19275, 4.00944, 5.05587, 8.09892,
                     9.47112, 4.84053, 11.2364, 10.0071], dtype=np.float32)
_A = 5
_CLS = 20
_G = 19
_N = _A * _G * _G
_NP = 2048
_B = 8
_CHUNK = 512
_NCHUNK = _NP // _CHUNK
_CONF_TH = 0.005
_NMS_TH = 0.45
_K = np.float32(_NMS_TH / (1.0 + _NMS_TH))


def _make_consts():
    i = np.arange(_NP)
    a = np.minimum(i // (_G * _G), _A - 1)
    p = i % (_G * _G)
    gx = (p % _G).astype(np.float32)
    gy = (p // _G).astype(np.float32)
    aw = _ANCHORS[2 * a]
    ah = _ANCHORS[2 * a + 1]
    return np.stack([gx, gy, aw, ah]).reshape(4, 1, _NP).astype(np.float32)


_CONSTS = _make_consts()


def _bitonic_stage(lane, k, j, key, idx, payloads):
    upper = (lane & j) == 0
    asc = (lane & k) == 0

    def partner(x):
        return jnp.where(upper, jnp.roll(x, -j, axis=1), jnp.roll(x, j, axis=1))

    pkey = partner(key)
    pidx = partner(idx)
    self_prec = (key > pkey) | ((key == pkey) & (idx < pidx))
    take = jnp.logical_xor(jnp.logical_xor(self_prec, upper),
                           jnp.logical_not(asc))
    new_key = jnp.where(take, pkey, key)
    new_idx = jnp.where(take, pidx, idx)
    new_pl = [jnp.where(take, partner(x), x) for x in payloads]
    return new_key, new_idx, new_pl


def _bitonic_sort(lane, key, idx, payloads):
    n = key.shape[1]
    k = 2
    while k <= n:
        j = k // 2
        while j >= 1:
            key, idx, payloads = _bitonic_stage(lane, k, j, key, idx, payloads)
            j //= 2
        k *= 2
    return key, idx, payloads


def _nms_body(o_ref, c_ref, cx_ref, cy_ref, w_ref, h_ref, p_ref):
    gx = c_ref[0]
    gy = c_ref[1]
    aw = c_ref[2]
    ah = c_ref[3]

    cx = (jax.nn.sigmoid(o_ref[0]) + gx) / np.float32(_G)
    cy = (jax.nn.sigmoid(o_ref[1]) + gy) / np.float32(_G)
    wv = jnp.exp(o_ref[2]) * aw / np.float32(_G)
    hv = jnp.exp(o_ref[3]) * ah / np.float32(_G)
    conf = jax.nn.sigmoid(o_ref[4])

    maxl = o_ref[5]
    for c in range(6, 5 + _CLS):
        maxl = jnp.maximum(maxl, o_ref[c])
    ssum = jnp.exp(o_ref[5] - maxl)
    for c in range(6, 5 + _CLS):
        ssum = ssum + jnp.exp(o_ref[c] - maxl)
    clsc = 1.0 / ssum

    lane = jax.lax.broadcasted_iota(jnp.int32, (_B, _NP), 1)
    valid = lane < _N
    wconf0 = jnp.where((conf > _CONF_TH) & valid, conf, -1.0)
    neg = np.float32(-3.4e38)

    skey, sidx, (scx, scy, swv, shv) = _bitonic_sort(
        lane, wconf0, lane, [cx, cy, wv, hv])

    sx1 = scx - swv / 2.0
    sx2 = scx + swv / 2.0
    sy1 = scy - shv / 2.0
    sy2 = scy + shv / 2.0
    ssa = (sx2 - sx1) * (sy2 - sy1) * _K

    kept_s = jnp.zeros((_B, _NP), jnp.float32)
    wc = skey
    lane_c = jax.lax.broadcasted_iota(jnp.int32, (_B, _CHUNK), 1)

    for c in range(_NCHUNK):
        base = c * _CHUNK
        S = _NP - base
        ax1 = sx1[:, base:]
        ax2 = sx2[:, base:]
        ay1 = sy1[:, base:]
        ay2 = sy2[:, base:]
        asa = ssa[:, base:]
        pcx = scx[:, base:base + _CHUNK]
        pcy = scy[:, base:base + _CHUNK]
        pw = swv[:, base:base + _CHUNK]
        ph = shv[:, base:base + _CHUNK]

        def step(wcs, kfs):
            ck = wcs[:, :_CHUNK]
            alive = ck > 0.0
            li = jnp.min(jnp.where(alive, lane_c, np.int32(2 ** 30)),
                         axis=1, keepdims=True)
            leader = (lane_c == li) & alive

            def ext(v):
                return jnp.max(jnp.where(leader, v, neg), axis=1, keepdims=True)

            lcx = ext(pcx)
            lcy = ext(pcy)
            lw = ext(pw)
            lh = ext(ph)
            lx1 = lcx - lw / 2.0
            lx2 = lcx + lw / 2.0
            ly1 = lcy - lh / 2.0
            ly2 = lcy + lh / 2.0
            lsa = (lx2 - lx1) * (ly2 - ly1) * _K
            iw = jnp.maximum(jnp.minimum(ax2, lx2) - jnp.maximum(ax1, lx1), 0.0)
            ih = jnp.maximum(jnp.minimum(ay2, ly2) - jnp.maximum(ay1, ly1), 0.0)
            keepalive = iw * ih <= asa + lsa
            new_wcs = jnp.where(keepalive, wcs, -1.0)
            new_ck = jnp.where(leader, -1.0, new_wcs[:, :_CHUNK])
            new_kf = jnp.maximum(kfs[:, :_CHUNK], leader.astype(jnp.float32))
            if S > _CHUNK:
                new_wcs = jnp.concatenate([new_ck, new_wcs[:, _CHUNK:]], axis=1)
                new_kfs = jnp.concatenate([new_kf, kfs[:, _CHUNK:]], axis=1)
            else:
                new_wcs = new_ck
                new_kfs = new_kf
            return new_wcs, new_kfs

        def ccond(st):
            wcs, _ = st
            return jnp.max(wcs[:, :_CHUNK]) > 0.0

        def cbody(st):
            wcs, kfs = st
            for _ in range(16):
                wcs, kfs = step(wcs, kfs)
            return wcs, kfs

        wc_suf, kept_suf = jax.lax.while_loop(
            ccond, cbody, (wc[:, base:], kept_s[:, base:]))
        if base > 0:
            wc = jnp.concatenate([wc[:, :base], wc_suf], axis=1)
            kept_s = jnp.concatenate([kept_s[:, :base], kept_suf], axis=1)
        else:
            wc = wc_suf
            kept_s = kept_suf

    ukey = (-sidx).astype(jnp.float32)
    _, _, (kept_orig,) = _bitonic_sort(lane, ukey, lane, [kept_s])
    prob = conf * clsc * kept_orig
    cx_ref[:, :] = cx
    cy_ref[:, :] = cy
    w_ref[:, :] = wv
    h_ref[:, :] = hv
    p_ref[:, :] = prob


def kernel(output):
    o = jnp.transpose(output.reshape(_B, _A, 5 + _CLS, _G * _G),
                      (2, 0, 1, 3)).reshape(5 + _CLS, _B, _N)
    o = jnp.pad(o, ((0, 0), (0, 0), (0, _NP - _N)))
    outs = pl.pallas_call(
        _nms_body,
        out_shape=[jax.ShapeDtypeStruct((_B, _NP), jnp.float32)] * 5,
    )(o, jnp.asarray(_CONSTS))
    return jnp.stack(outs, axis=-1)[:, :_N, :]
